# baseline (speedup 1.0000x reference)
import jax
import jax.numpy as jnp
from jax.experimental import pallas as pl
from jax.experimental.pallas import tpu as pltpu

_BM = 512


def _mm_relu(x_ref, w_hbm, o_ref, wf0_ref, wf1_ref, wb_ref, sem0, sem1):
    K, N = w_hbm.shape
    nq = N // 4
    i = pl.program_id(0)

    @pl.when(i == 0)
    def _():
        stage = (wf0_ref, wf1_ref)
        sems = (sem0, sem1)

        def _copy(q):
            cp = pltpu.make_async_copy(
                w_hbm.at[:, pl.ds(q * nq, nq)], stage[q % 2], sems[q % 2])
            cp.start()
            return cp

        cps = [_copy(0), _copy(1)]
        xb = x_ref[...].astype(jnp.bfloat16)
        for q in range(4):
            sl = pl.ds(q * nq, nq)
            cps[q].wait()
            wb_ref[:, sl] = stage[q % 2][...].astype(jnp.bfloat16)
            if q + 2 < 4:
                cps.append(_copy(q + 2))
            acc = jnp.dot(xb, wb_ref[:, sl], preferred_element_type=jnp.float32)
            o_ref[:, sl] = jnp.maximum(acc, 0.0)

    @pl.when(i > 0)
    def _():
        xb = x_ref[...].astype(jnp.bfloat16)
        acc = jnp.dot(xb, wb_ref[...], preferred_element_type=jnp.float32)
        o_ref[...] = jnp.maximum(acc, 0.0)


def kernel(x_affine, W):
    M, K = x_affine.shape
    _, N = W.shape
    return pl.pallas_call(
        _mm_relu,
        grid=(M // _BM,),
        in_specs=[
            pl.BlockSpec((_BM, K), lambda i: (i, 0)),
            pl.BlockSpec(memory_space=pl.ANY),
        ],
        out_specs=pl.BlockSpec((_BM, N), lambda i: (i, 0)),
        out_shape=jax.ShapeDtypeStruct((M, N), jnp.float32),
        scratch_shapes=[
            pltpu.VMEM((K, N // 4), jnp.float32),
            pltpu.VMEM((K, N // 4), jnp.float32),
            pltpu.VMEM((K, N), jnp.bfloat16),
            pltpu.SemaphoreType.DMA,
            pltpu.SemaphoreType.DMA,
        ],
        compiler_params=pltpu.CompilerParams(
            dimension_semantics=("arbitrary",),
        ),
    )(x_affine, W)


# speedup vs baseline: 1.0182x; 1.0182x over previous
import jax
import jax.numpy as jnp
from jax.experimental import pallas as pl
from jax.experimental.pallas import tpu as pltpu

_BM = 512


def _mm_relu(x_ref, w_hbm, o_ref, wf0_ref, wf1_ref, wb_ref, sem0, sem1):
    K, N = w_hbm.shape
    kh, nh = K // 2, N // 2
    i = pl.program_id(0)

    @pl.when(i == 0)
    def _():
        stage = (wf0_ref, wf1_ref)
        sems = (sem0, sem1)

        def _copy(j):
            kj, nj = j % 2, j // 2
            cp = pltpu.make_async_copy(
                w_hbm.at[pl.ds(kj * kh, kh), pl.ds(nj * nh, nh)],
                stage[j % 2], sems[j % 2])
            cp.start()
            return cp

        cps = [_copy(0), _copy(1)]
        xb = x_ref[...].astype(jnp.bfloat16)
        for nj in range(2):
            nsl = pl.ds(nj * nh, nh)
            acc = None
            for kj in range(2):
                j = 2 * nj + kj
                ksl = pl.ds(kj * kh, kh)
                cps[j].wait()
                wb_ref[ksl, nsl] = stage[j % 2][...].astype(jnp.bfloat16)
                if j + 2 < 4:
                    cps.append(_copy(j + 2))
                part = jnp.dot(xb[:, kj * kh : kj * kh + kh], wb_ref[ksl, nsl],
                               preferred_element_type=jnp.float32)
                acc = part if acc is None else acc + part
            o_ref[:, nsl] = jnp.maximum(acc, 0.0)

    @pl.when(i > 0)
    def _():
        xb = x_ref[...].astype(jnp.bfloat16)
        acc = jnp.dot(xb, wb_ref[...], preferred_element_type=jnp.float32)
        o_ref[...] = jnp.maximum(acc, 0.0)


def kernel(x_affine, W):
    M, K = x_affine.shape
    _, N = W.shape
    return pl.pallas_call(
        _mm_relu,
        grid=(M // _BM,),
        in_specs=[
            pl.BlockSpec((_BM, K), lambda i: (i, 0)),
            pl.BlockSpec(memory_space=pl.ANY),
        ],
        out_specs=pl.BlockSpec((_BM, N), lambda i: (i, 0)),
        out_shape=jax.ShapeDtypeStruct((M, N), jnp.float32),
        scratch_shapes=[
            pltpu.VMEM((K // 2, N // 2), jnp.float32),
            pltpu.VMEM((K // 2, N // 2), jnp.float32),
            pltpu.VMEM((K, N), jnp.bfloat16),
            pltpu.SemaphoreType.DMA,
            pltpu.SemaphoreType.DMA,
        ],
        compiler_params=pltpu.CompilerParams(
            dimension_semantics=("arbitrary",),
        ),
    )(x_affine, W)
